# baseline (speedup 1.0000x reference)
import math

import jax
import jax.numpy as jnp
import numpy as np
from jax.experimental import pallas as pl
from jax.experimental.pallas import tpu as pltpu

_SIZE = 100000
_CONF = 0.9
_EPS = float(np.float32(0.1 / (_SIZE - 2)))
_H = _CONF * math.log(_CONF) + (_SIZE - 2) * _EPS * math.log(_EPS)
_ROWS_PER_BLOCK = 32


def _tc_body(t_ref, x_ref, o_ref):
    pid = pl.program_id(0)
    x = x_ref[...]
    t = t_ref[...]
    w = (t != 0).astype(jnp.float32)
    iota = jax.lax.broadcasted_iota(jnp.int32, x.shape, 1)
    rowsum = jnp.sum(x, axis=1, keepdims=True)
    vk = jnp.sum(jnp.where(iota == t, x, 0.0), axis=1, keepdims=True)
    contrib = jnp.sum(
        w * (_H - (_CONF - _EPS) * vk - _EPS * (rowsum - x[:, 0:1]))
    )

    @pl.when(pid == 0)
    def _init():
        o_ref[0, 0] = 0.0

    o_ref[0, 0] += contrib


def kernel(x, target):
    n = x.shape[0]
    r = _ROWS_PER_BLOCK
    t2d = target.astype(jnp.int32).reshape(n, 1)
    out = pl.pallas_call(
        _tc_body,
        grid=(n // r,),
        in_specs=[
            pl.BlockSpec((r, 1), lambda i: (i, 0)),
            pl.BlockSpec((r, _SIZE), lambda i: (i, 0)),
        ],
        out_specs=pl.BlockSpec(memory_space=pltpu.SMEM),
        out_shape=jax.ShapeDtypeStruct((1, 1), jnp.float32),
    )(t2d, x)
    return out[0, 0]
